# baseline (speedup 1.0000x reference)
import functools

import jax
import jax.numpy as jnp
from jax import lax
from jax.experimental import pallas as pl
from jax.experimental.pallas import tpu as pltpu
from jax.experimental.pallas import tpu_sc as plsc

_L = 16


def _make_permute(B, N, rows_per_chunk):
    info = plsc.get_sparse_core_info()
    nc, ns = info.num_cores, info.num_subcores
    nw = nc * ns
    assert B % (nw * rows_per_chunk) == 0
    rows_per_w = B // nw
    chunks = rows_per_w // rows_per_chunk
    groups = N // _L
    chunk_elems = rows_per_chunk * N

    mesh = plsc.VectorSubcoreMesh(core_axis_name="c", subcore_axis_name="s")

    @functools.partial(
        pl.kernel,
        mesh=mesh,
        out_type=jax.ShapeDtypeStruct((B * N,), jnp.float32),
        compiler_params=pltpu.CompilerParams(needs_layout_passes=False),
        scratch_types=[
            pltpu.VMEM((N,), jnp.int32),
            pltpu.VMEM((chunk_elems,), jnp.float32),
            pltpu.VMEM((chunk_elems,), jnp.float32),
        ],
    )
    def permute(y_hbm, idx_hbm, out_hbm, idx_v, in_v, out_v):
        wid = lax.axis_index("s") * nc + lax.axis_index("c")
        base = wid * rows_per_w * N
        pltpu.sync_copy(idx_hbm, idx_v)

        def do_chunk(c, carry):
            off0 = base + c * chunk_elems
            pltpu.sync_copy(y_hbm.at[pl.ds(off0, chunk_elems)], in_v)

            def do_group(j, carry2):
                col = j * _L
                idxv = idx_v[pl.ds(col, _L)]
                for r in range(rows_per_chunk):
                    out_v[pl.ds(col + r * N, _L)] = plsc.load_gather(
                        in_v, [idxv + (r * N)]
                    )
                return carry2

            lax.fori_loop(0, groups, do_group, 0, unroll=False)
            pltpu.sync_copy(out_v, out_hbm.at[pl.ds(off0, chunk_elems)])
            return carry

        lax.fori_loop(0, chunks, do_chunk, 0, unroll=False)

    return permute


def kernel(y, indices, indices_inverse):
    B, N = y.shape
    fn = _make_permute(B, N, rows_per_chunk=8)
    out = fn(y.reshape(-1), indices.astype(jnp.int32))
    return out.reshape(B, N)


# speedup vs baseline: 1.1296x; 1.1296x over previous
import functools

import jax
import jax.numpy as jnp
from jax import lax
from jax.experimental import pallas as pl
from jax.experimental.pallas import tpu as pltpu
from jax.experimental.pallas import tpu_sc as plsc

_L = 16


def _make_permute(B, N, rows_per_chunk, unroll):
    info = plsc.get_sparse_core_info()
    nc, ns = info.num_cores, info.num_subcores
    nw = nc * ns
    assert B % (nw * 2 * rows_per_chunk) == 0
    rows_per_w = B // nw
    chunks = rows_per_w // rows_per_chunk
    pairs = chunks // 2
    groups = N // _L
    chunk_elems = rows_per_chunk * N

    mesh = plsc.VectorSubcoreMesh(core_axis_name="c", subcore_axis_name="s")

    @functools.partial(
        pl.kernel,
        mesh=mesh,
        out_type=jax.ShapeDtypeStruct((B * N,), jnp.float32),
        compiler_params=pltpu.CompilerParams(needs_layout_passes=False),
        scratch_types=[
            pltpu.VMEM((N,), jnp.int32),
            pltpu.VMEM((chunk_elems,), jnp.float32),
            pltpu.VMEM((chunk_elems,), jnp.float32),
            pltpu.VMEM((chunk_elems,), jnp.float32),
            pltpu.VMEM((chunk_elems,), jnp.float32),
            pltpu.SemaphoreType.DMA,
            pltpu.SemaphoreType.DMA,
            pltpu.SemaphoreType.DMA,
            pltpu.SemaphoreType.DMA,
        ],
    )
    def permute(
        y_hbm, idx_hbm, out_hbm,
        idx_v, in0, in1, out0, out1, si0, si1, so0, so1,
    ):
        wid = lax.axis_index("s") * nc + lax.axis_index("c")
        base = wid * rows_per_w * N
        pltpu.sync_copy(idx_hbm, idx_v)

        def in_src(c):
            return y_hbm.at[pl.ds(base + c * chunk_elems, chunk_elems)]

        def out_dst(c):
            return out_hbm.at[pl.ds(base + c * chunk_elems, chunk_elems)]

        def compute(in_b, out_b):
            def _grp(j, carry2):
                col = j * _L
                idxv = idx_v[pl.ds(col, _L)]
                for r in range(rows_per_chunk):
                    out_b[pl.ds(col + r * N, _L)] = plsc.load_gather(
                        in_b, [idxv + (r * N)]
                    )
                return carry2

            lax.fori_loop(0, groups, _grp, 0, unroll=False)

        pltpu.async_copy(in_src(0), in0, si0)

        def do_pair(p, carry):
            c0 = p * 2

            pltpu.async_copy(in_src(c0 + 1), in1, si1)
            pltpu.make_async_copy(in_src(c0), in0, si0).wait()

            @pl.when(p > 0)
            def _():
                pltpu.make_async_copy(out0, out_dst(c0), so0).wait()

            compute(in0, out0)
            pltpu.async_copy(out0, out_dst(c0), so0)

            @pl.when(p + 1 < pairs)
            def _():
                pltpu.async_copy(in_src(c0 + 2), in0, si0)

            pltpu.make_async_copy(in_src(c0 + 1), in1, si1).wait()

            @pl.when(p > 0)
            def _():
                pltpu.make_async_copy(out1, out_dst(c0 + 1), so1).wait()

            compute(in1, out1)
            pltpu.async_copy(out1, out_dst(c0 + 1), so1)
            return carry

        lax.fori_loop(0, pairs, do_pair, 0, unroll=False)

        pltpu.make_async_copy(out0, out_dst(chunks - 2), so0).wait()
        pltpu.make_async_copy(out1, out_dst(chunks - 1), so1).wait()

    return permute


def kernel(y, indices, indices_inverse):
    B, N = y.shape
    fn = _make_permute(B, N, rows_per_chunk=4, unroll=4)
    out = fn(y.reshape(-1), indices.astype(jnp.int32))
    return out.reshape(B, N)


# speedup vs baseline: 2.0186x; 1.7870x over previous
import functools

import jax
import jax.numpy as jnp
from jax import lax
from jax.experimental import pallas as pl
from jax.experimental.pallas import tpu as pltpu
from jax.experimental.pallas import tpu_sc as plsc

_L = 16


def _make_permute(B, N, rows_per_chunk, unroll):
    info = plsc.get_sparse_core_info()
    nc, ns = info.num_cores, info.num_subcores
    nw = nc * ns
    assert B % (nw * 2 * rows_per_chunk) == 0
    rows_per_w = B // nw
    chunks = rows_per_w // rows_per_chunk
    pairs = chunks // 2
    groups = N // _L
    chunk_elems = rows_per_chunk * N

    mesh = plsc.VectorSubcoreMesh(core_axis_name="c", subcore_axis_name="s")

    @functools.partial(
        pl.kernel,
        mesh=mesh,
        out_type=jax.ShapeDtypeStruct((B * N,), jnp.float32),
        compiler_params=pltpu.CompilerParams(needs_layout_passes=False),
        scratch_types=[
            pltpu.VMEM((N,), jnp.int32),
            pltpu.VMEM((chunk_elems,), jnp.float32),
            pltpu.VMEM((chunk_elems,), jnp.float32),
            pltpu.VMEM((chunk_elems,), jnp.float32),
            pltpu.VMEM((chunk_elems,), jnp.float32),
            pltpu.SemaphoreType.DMA,
            pltpu.SemaphoreType.DMA,
            pltpu.SemaphoreType.DMA,
            pltpu.SemaphoreType.DMA,
        ],
    )
    def permute(
        y_hbm, idx_hbm, out_hbm,
        idx_v, in0, in1, out0, out1, si0, si1, so0, so1,
    ):
        wid = lax.axis_index("s") * nc + lax.axis_index("c")
        base = wid * rows_per_w * N
        pltpu.sync_copy(idx_hbm, idx_v)

        def in_src(c):
            return y_hbm.at[pl.ds(base + c * chunk_elems, chunk_elems)]

        def out_dst(c):
            return out_hbm.at[pl.ds(base + c * chunk_elems, chunk_elems)]

        def compute(in_b, out_b):
            @plsc.parallel_loop(0, groups, unroll=unroll)
            def _(j):
                col = j * _L
                idxv = idx_v[pl.ds(col, _L)]
                for r in range(rows_per_chunk):
                    out_b[pl.ds(col + r * N, _L)] = plsc.load_gather(
                        in_b, [idxv + (r * N)]
                    )

        pltpu.async_copy(in_src(0), in0, si0)

        def do_pair(p, carry):
            c0 = p * 2

            pltpu.async_copy(in_src(c0 + 1), in1, si1)
            pltpu.make_async_copy(in_src(c0), in0, si0).wait()

            @pl.when(p > 0)
            def _():
                pltpu.make_async_copy(out0, out_dst(c0), so0).wait()

            compute(in0, out0)
            pltpu.async_copy(out0, out_dst(c0), so0)

            @pl.when(p + 1 < pairs)
            def _():
                pltpu.async_copy(in_src(c0 + 2), in0, si0)

            pltpu.make_async_copy(in_src(c0 + 1), in1, si1).wait()

            @pl.when(p > 0)
            def _():
                pltpu.make_async_copy(out1, out_dst(c0 + 1), so1).wait()

            compute(in1, out1)
            pltpu.async_copy(out1, out_dst(c0 + 1), so1)
            return carry

        lax.fori_loop(0, pairs, do_pair, 0, unroll=False)

        pltpu.make_async_copy(out0, out_dst(chunks - 2), so0).wait()
        pltpu.make_async_copy(out1, out_dst(chunks - 1), so1).wait()

    return permute


def kernel(y, indices, indices_inverse):
    B, N = y.shape
    fn = _make_permute(B, N, rows_per_chunk=4, unroll=4)
    out = fn(y.reshape(-1), indices.astype(jnp.int32))
    return out.reshape(B, N)


# speedup vs baseline: 2.0224x; 1.0019x over previous
import functools

import jax
import jax.numpy as jnp
from jax import lax
from jax.experimental import pallas as pl
from jax.experimental.pallas import tpu as pltpu
from jax.experimental.pallas import tpu_sc as plsc

_L = 16


def _make_permute(B, N, rows_per_chunk, unroll):
    info = plsc.get_sparse_core_info()
    nc, ns = info.num_cores, info.num_subcores
    nw = nc * ns
    assert B % (nw * 2 * rows_per_chunk) == 0
    rows_per_w = B // nw
    chunks = rows_per_w // rows_per_chunk
    pairs = chunks // 2
    groups = N // _L
    chunk_elems = rows_per_chunk * N

    mesh = plsc.VectorSubcoreMesh(core_axis_name="c", subcore_axis_name="s")

    @functools.partial(
        pl.kernel,
        mesh=mesh,
        out_type=jax.ShapeDtypeStruct((B * N,), jnp.float32),
        compiler_params=pltpu.CompilerParams(needs_layout_passes=False),
        scratch_types=[
            pltpu.VMEM((N,), jnp.int32),
            pltpu.VMEM((chunk_elems,), jnp.float32),
            pltpu.VMEM((chunk_elems,), jnp.float32),
            pltpu.VMEM((chunk_elems,), jnp.float32),
            pltpu.VMEM((chunk_elems,), jnp.float32),
            pltpu.SemaphoreType.DMA,
            pltpu.SemaphoreType.DMA,
            pltpu.SemaphoreType.DMA,
            pltpu.SemaphoreType.DMA,
        ],
    )
    def permute(
        y_hbm, idx_hbm, out_hbm,
        idx_v, in0, in1, out0, out1, si0, si1, so0, so1,
    ):
        wid = lax.axis_index("s") * nc + lax.axis_index("c")
        base = wid * rows_per_w * N
        pltpu.sync_copy(idx_hbm, idx_v)

        def in_src(c):
            return y_hbm.at[pl.ds(base + c * chunk_elems, chunk_elems)]

        def out_dst(c):
            return out_hbm.at[pl.ds(base + c * chunk_elems, chunk_elems)]

        def compute(in_b, out_b):
            @plsc.parallel_loop(0, groups, unroll=unroll)
            def _(j):
                col = j * _L
                idxv = idx_v[pl.ds(col, _L)]
                for r in range(rows_per_chunk):
                    out_b[pl.ds(col + r * N, _L)] = plsc.load_gather(
                        in_b, [idxv + (r * N)]
                    )

        pltpu.async_copy(in_src(0), in0, si0)

        def do_pair(p, carry):
            c0 = p * 2

            pltpu.async_copy(in_src(c0 + 1), in1, si1)
            pltpu.make_async_copy(in_src(c0), in0, si0).wait()

            @pl.when(p > 0)
            def _():
                pltpu.make_async_copy(out0, out_dst(c0), so0).wait()

            compute(in0, out0)
            pltpu.async_copy(out0, out_dst(c0), so0)

            @pl.when(p + 1 < pairs)
            def _():
                pltpu.async_copy(in_src(c0 + 2), in0, si0)

            pltpu.make_async_copy(in_src(c0 + 1), in1, si1).wait()

            @pl.when(p > 0)
            def _():
                pltpu.make_async_copy(out1, out_dst(c0 + 1), so1).wait()

            compute(in1, out1)
            pltpu.async_copy(out1, out_dst(c0 + 1), so1)
            return carry

        lax.fori_loop(0, pairs, do_pair, 0, unroll=False)

        pltpu.make_async_copy(out0, out_dst(chunks - 2), so0).wait()
        pltpu.make_async_copy(out1, out_dst(chunks - 1), so1).wait()

    return permute


def kernel(y, indices, indices_inverse):
    B, N = y.shape
    fn = _make_permute(B, N, rows_per_chunk=4, unroll=8)
    out = fn(y.reshape(-1), indices.astype(jnp.int32))
    return out.reshape(B, N)


# speedup vs baseline: 6.2764x; 3.1034x over previous
import functools

import jax
import jax.numpy as jnp
from jax import lax
from jax.experimental import pallas as pl
from jax.experimental.pallas import tpu as pltpu
from jax.experimental.pallas import tpu_sc as plsc

_L = 16


def _make_permute(B, N, rows_per_chunk, unroll):
    info = plsc.get_sparse_core_info()
    nc, ns = info.num_cores, info.num_subcores
    nw = nc * ns
    assert B % (nw * 2 * rows_per_chunk) == 0
    rows_per_w = B // nw
    chunks = rows_per_w // rows_per_chunk
    pairs = chunks // 2
    groups = N // _L

    mesh = plsc.VectorSubcoreMesh(core_axis_name="c", subcore_axis_name="s")

    @functools.partial(
        pl.kernel,
        mesh=mesh,
        out_type=jax.ShapeDtypeStruct((B, N), jnp.float32),
        compiler_params=pltpu.CompilerParams(needs_layout_passes=False),
        scratch_types=[
            pltpu.VMEM((N,), jnp.int32),
            pltpu.VMEM((rows_per_chunk, N), jnp.float32),
            pltpu.VMEM((rows_per_chunk, N), jnp.float32),
            pltpu.VMEM((rows_per_chunk, N), jnp.float32),
            pltpu.VMEM((rows_per_chunk, N), jnp.float32),
            pltpu.SemaphoreType.DMA,
            pltpu.SemaphoreType.DMA,
            pltpu.SemaphoreType.DMA,
            pltpu.SemaphoreType.DMA,
        ],
    )
    def permute(
        y_hbm, idx_hbm, out_hbm,
        idx_v, in0, in1, out0, out1, si0, si1, so0, so1,
    ):
        wid = lax.axis_index("s") * nc + lax.axis_index("c")
        base = wid * rows_per_w
        pltpu.sync_copy(idx_hbm, idx_v)

        def in_src(c):
            return y_hbm.at[pl.ds(base + c * rows_per_chunk, rows_per_chunk)]

        def out_dst(c):
            return out_hbm.at[pl.ds(base + c * rows_per_chunk, rows_per_chunk)]

        def compute(in_b, out_b):
            @plsc.parallel_loop(0, groups, unroll=unroll)
            def _(j):
                col = j * _L
                idxv = idx_v[pl.ds(col, _L)]
                for r in range(rows_per_chunk):
                    rvec = jnp.full((_L,), r, jnp.int32)
                    out_b[r, pl.ds(col, _L)] = plsc.load_gather(
                        in_b, [rvec, idxv]
                    )

        pltpu.async_copy(in_src(0), in0, si0)

        def do_pair(p, carry):
            c0 = p * 2

            pltpu.async_copy(in_src(c0 + 1), in1, si1)
            pltpu.make_async_copy(in_src(c0), in0, si0).wait()

            @pl.when(p > 0)
            def _():
                pltpu.make_async_copy(out0, out_dst(c0), so0).wait()

            compute(in0, out0)
            pltpu.async_copy(out0, out_dst(c0), so0)

            @pl.when(p + 1 < pairs)
            def _():
                pltpu.async_copy(in_src(c0 + 2), in0, si0)

            pltpu.make_async_copy(in_src(c0 + 1), in1, si1).wait()

            @pl.when(p > 0)
            def _():
                pltpu.make_async_copy(out1, out_dst(c0 + 1), so1).wait()

            compute(in1, out1)
            pltpu.async_copy(out1, out_dst(c0 + 1), so1)
            return carry

        lax.fori_loop(0, pairs, do_pair, 0, unroll=False)

        pltpu.make_async_copy(out0, out_dst(chunks - 2), so0).wait()
        pltpu.make_async_copy(out1, out_dst(chunks - 1), so1).wait()

    return permute


def kernel(y, indices, indices_inverse):
    B, N = y.shape
    fn = _make_permute(B, N, rows_per_chunk=4, unroll=8)
    return fn(y, indices.astype(jnp.int32))


# speedup vs baseline: 6.4563x; 1.0287x over previous
import functools

import jax
import jax.numpy as jnp
from jax import lax
from jax.experimental import pallas as pl
from jax.experimental.pallas import tpu as pltpu
from jax.experimental.pallas import tpu_sc as plsc

_L = 16


def _make_permute(B, N, rows_per_chunk, unroll):
    info = plsc.get_sparse_core_info()
    nc, ns = info.num_cores, info.num_subcores
    nw = nc * ns
    assert B % (nw * 4 * rows_per_chunk) == 0
    rows_per_w = B // nw
    chunks = rows_per_w // rows_per_chunk
    quads = chunks // 4
    groups = N // _L

    mesh = plsc.VectorSubcoreMesh(core_axis_name="c", subcore_axis_name="s")

    @functools.partial(
        pl.kernel,
        mesh=mesh,
        out_type=jax.ShapeDtypeStruct((B, N), jnp.float32),
        compiler_params=pltpu.CompilerParams(needs_layout_passes=False),
        scratch_types=[
            pltpu.VMEM((N,), jnp.int32),
            pltpu.VMEM((rows_per_chunk, N), jnp.float32),
            pltpu.VMEM((rows_per_chunk, N), jnp.float32),
            pltpu.VMEM((rows_per_chunk, N), jnp.float32),
            pltpu.VMEM((rows_per_chunk, N), jnp.float32),
            pltpu.VMEM((rows_per_chunk, N), jnp.float32),
            pltpu.VMEM((rows_per_chunk, N), jnp.float32),
            pltpu.SemaphoreType.DMA,
            pltpu.SemaphoreType.DMA,
            pltpu.SemaphoreType.DMA,
            pltpu.SemaphoreType.DMA,
            pltpu.SemaphoreType.DMA,
            pltpu.SemaphoreType.DMA,
        ],
    )
    def permute(
        y_hbm, idx_hbm, out_hbm,
        idx_v, ina, inb, inc, ind, out0, out1,
        sa, sb, sc, sd, so0, so1,
    ):
        wid = lax.axis_index("s") * nc + lax.axis_index("c")
        base = wid * rows_per_w
        pltpu.sync_copy(idx_hbm, idx_v)

        def in_src(c):
            return y_hbm.at[pl.ds(base + c * rows_per_chunk, rows_per_chunk)]

        def out_dst(c):
            return out_hbm.at[pl.ds(base + c * rows_per_chunk, rows_per_chunk)]

        def compute(in_b, out_b):
            @plsc.parallel_loop(0, groups, unroll=unroll)
            def _(j):
                col = j * _L
                idxv = idx_v[pl.ds(col, _L)]
                for r in range(rows_per_chunk):
                    rvec = jnp.full((_L,), r, jnp.int32)
                    out_b[r, pl.ds(col, _L)] = plsc.load_gather(
                        in_b, [rvec, idxv]
                    )

        pltpu.async_copy(in_src(0), ina, sa)
        pltpu.async_copy(in_src(1), inb, sb)
        pltpu.async_copy(in_src(2), inc, sc)

        in_bufs = (ina, inb, inc, ind)
        in_sems = (sa, sb, sc, sd)
        out_bufs = (out0, out1)
        out_sems = (so0, so1)

        def do_quad(q, carry):
            c0 = q * 4
            for k in range(4):
                c = c0 + k
                pre = (k + 3) % 4

                @pl.when(c + 3 < chunks)
                def _():
                    pltpu.async_copy(in_src(c + 3), in_bufs[pre], in_sems[pre])

                pltpu.make_async_copy(
                    in_src(c), in_bufs[k], in_sems[k]
                ).wait()

                ob = out_bufs[k % 2]
                os = out_sems[k % 2]

                @pl.when(c > 1)
                def _():
                    pltpu.make_async_copy(ob, out_dst(c - 2), os).wait()

                compute(in_bufs[k], ob)
                pltpu.async_copy(ob, out_dst(c), os)
            return carry

        lax.fori_loop(0, quads, do_quad, 0, unroll=False)

        pltpu.make_async_copy(out0, out_dst(chunks - 2), so0).wait()
        pltpu.make_async_copy(out1, out_dst(chunks - 1), so1).wait()

    return permute


def kernel(y, indices, indices_inverse):
    B, N = y.shape
    fn = _make_permute(B, N, rows_per_chunk=4, unroll=8)
    return fn(y, indices.astype(jnp.int32))
